# baseline (speedup 1.0000x reference)
import jax
import jax.numpy as jnp
from jax import lax
from jax.experimental import pallas as pl
from jax.experimental.pallas import tpu as pltpu
from jax.experimental.pallas import tpu_sc as plsc

N_ROWS = 1_000_000
D_COLS = 64
Q = 16384
D_SEL = 32

PAIR_ROWS = N_ROWS // 2
PAIR_W = 2 * D_COLS

NUM_CORES = 2
NUM_SUBCORES = 16
NW = NUM_CORES * NUM_SUBCORES
BPW = Q // NW
CHUNK = 128
NCHUNK = BPW // CHUNK
LANES = 16


def _query_body(table_hbm, rows_hbm, cols_hbm, out_hbm,
                raw_v, idx_v, off_v, cols_v, rows_v, out_v, sem):
    wid = lax.axis_index("s") * NUM_CORES + lax.axis_index("c")
    base = wid * BPW

    for j in range(NCHUNK):
        pltpu.sync_copy(rows_hbm.at[pl.ds(base + j * CHUNK, CHUNK)],
                        raw_v.at[j])
    pltpu.sync_copy(cols_hbm, cols_v)

    def shift_body(g, carry):
        j = g // (CHUNK // LANES)
        o = (g % (CHUNK // LANES)) * LANES
        raw = raw_v[j, pl.ds(o, LANES)]
        idx_v[j, pl.ds(o, LANES)] = lax.shift_right_logical(raw, 1)
        off_v[j, pl.ds(o, LANES)] = lax.shift_left(
            lax.bitwise_and(raw, 1), 6)
        return carry

    lax.fori_loop(0, NCHUNK * (CHUNK // LANES), shift_body, 0, unroll=True)

    copies = []
    for j in range(NCHUNK):
        copies.append(pltpu.async_copy(
            table_hbm.at[idx_v.at[j]],
            rows_v.at[pl.ds(j * CHUNK, CHUNK)],
            sem))
    for c in copies:
        c.wait()

    c_lo = cols_v[pl.ds(0, LANES)]
    c_hi = cols_v[pl.ds(LANES, LANES)]

    def body(r, carry):
        jv = jnp.full((LANES,), r // CHUNK, dtype=jnp.int32)
        ov = jnp.full((LANES,), r % CHUNK, dtype=jnp.int32)
        off = plsc.load_gather(off_v, [jv, ov])
        ridx = jnp.full((LANES,), r, dtype=jnp.int32)
        po = (r % 4) * D_SEL
        out_v[r // 4, pl.ds(po, LANES)] = plsc.load_gather(
            rows_v, [ridx, c_lo + off])
        out_v[r // 4, pl.ds(po + LANES, LANES)] = plsc.load_gather(
            rows_v, [ridx, c_hi + off])
        return carry

    lax.fori_loop(0, BPW, body, 0, unroll=4)

    out_base = pl.multiple_of(base // 4, PAIR_W)
    pltpu.sync_copy(out_v, out_hbm.at[pl.ds(out_base, BPW // 4)])


@jax.jit
def kernel(table, rows, cols):
    mesh = plsc.VectorSubcoreMesh(
        core_axis_name="c", subcore_axis_name="s",
        num_cores=NUM_CORES, num_subcores=NUM_SUBCORES)
    run = pl.kernel(
        _query_body,
        out_type=jax.ShapeDtypeStruct((Q * D_SEL // PAIR_W, PAIR_W),
                                      jnp.float32),
        mesh=mesh,
        scratch_types=[
            pltpu.VMEM((NCHUNK, CHUNK), jnp.int32),
            pltpu.VMEM((NCHUNK, CHUNK), jnp.int32),
            pltpu.VMEM((NCHUNK, CHUNK), jnp.int32),
            pltpu.VMEM((D_SEL,), jnp.int32),
            pltpu.VMEM((BPW, PAIR_W), jnp.float32),
            pltpu.VMEM((BPW // 4, PAIR_W), jnp.float32),
            pltpu.SemaphoreType.DMA,
        ],
        compiler_params=pltpu.CompilerParams(
            needs_layout_passes=False, use_tc_tiling_on_sc=True),
    )
    table_pairs = table.reshape(PAIR_ROWS, PAIR_W)
    out = run(table_pairs, rows.astype(jnp.int32), cols.astype(jnp.int32))
    return out.reshape(Q, D_SEL)


# speedup vs baseline: 1.7125x; 1.7125x over previous
import jax
import jax.numpy as jnp
from jax import lax
from jax.experimental import pallas as pl
from jax.experimental.pallas import tpu as pltpu
from jax.experimental.pallas import tpu_sc as plsc

N_ROWS = 1_000_000
D_COLS = 64
Q = 16384
D_SEL = 32

NUM_CORES = 2
NUM_SUBCORES = 16
NW = NUM_CORES * NUM_SUBCORES
BPW = Q // NW
LANES = 16
OUT_W = 4 * D_SEL


def _query_body(table_hbm, rows_hbm, cols_hbm, out_hbm,
                idx_v, cols_v, rows_v, out_v, sem):
    wid = lax.axis_index("s") * NUM_CORES + lax.axis_index("c")
    base = wid * BPW

    pltpu.sync_copy(rows_hbm.at[pl.ds(base, BPW)], idx_v)
    pltpu.sync_copy(cols_hbm, cols_v)

    def fire(g, carry):
        v = idx_v[pl.ds(g * LANES, LANES)]
        for k in range(LANES):
            pltpu.async_copy(table_hbm.at[pl.ds(v[k], 1)],
                             rows_v.at[pl.ds(g * LANES + k, 1)],
                             sem)
        return carry

    lax.fori_loop(0, BPW // LANES, fire, 0)

    pltpu.make_async_copy(table_hbm.at[pl.ds(0, BPW)], rows_v, sem).wait()

    c_lo = cols_v[pl.ds(0, LANES)]
    c_hi = cols_v[pl.ds(LANES, LANES)]

    def body(r, carry):
        ridx = jnp.full((LANES,), r, dtype=jnp.int32)
        po = (r % 4) * D_SEL
        out_v[r // 4, pl.ds(po, LANES)] = plsc.load_gather(
            rows_v, [ridx, c_lo])
        out_v[r // 4, pl.ds(po + LANES, LANES)] = plsc.load_gather(
            rows_v, [ridx, c_hi])
        return carry

    lax.fori_loop(0, BPW, body, 0, unroll=4)

    out_base = pl.multiple_of(base // 4, OUT_W)
    pltpu.sync_copy(out_v, out_hbm.at[pl.ds(out_base, BPW // 4)])


@jax.jit
def kernel(table, rows, cols):
    mesh = plsc.VectorSubcoreMesh(
        core_axis_name="c", subcore_axis_name="s",
        num_cores=NUM_CORES, num_subcores=NUM_SUBCORES)
    run = pl.kernel(
        _query_body,
        out_type=jax.ShapeDtypeStruct((Q * D_SEL // OUT_W, OUT_W),
                                      jnp.float32),
        mesh=mesh,
        scratch_types=[
            pltpu.VMEM((BPW,), jnp.int32),
            pltpu.VMEM((D_SEL,), jnp.int32),
            pltpu.VMEM((BPW, D_COLS), jnp.float32),
            pltpu.VMEM((BPW // 4, OUT_W), jnp.float32),
            pltpu.SemaphoreType.DMA,
        ],
        compiler_params=pltpu.CompilerParams(
            needs_layout_passes=False, use_tc_tiling_on_sc=True),
    )
    out = run(table, rows.astype(jnp.int32), cols.astype(jnp.int32))
    return out.reshape(Q, D_SEL)
